# Initial kernel scaffold; baseline (speedup 1.0000x reference)
import jax
import jax.numpy as jnp
from jax.experimental import pallas as pl


def kernel(x, x_proj_weight, dt_projs_weight, dt_projs_bias, A_logs, Ds, h_norm_w, h_norm_b, out_norm_w, out_norm_b):
    raise NotImplementedError("write your pallas kernel here")


# speedup vs baseline: 407.3581x; 407.3581x over previous
import jax
import jax.numpy as jnp
from jax import lax
from jax.experimental import pallas as pl


def _shift(x, axis, s, forward, identity):
    n = x.shape[axis]
    if forward:
        body = lax.slice_in_dim(x, 0, n - s, axis=axis)
        pad_shape = list(x.shape)
        pad_shape[axis] = s
        pad = jnp.full(pad_shape, identity, dtype=x.dtype)
        return jnp.concatenate([pad, body], axis=axis)
    else:
        body = lax.slice_in_dim(x, s, n, axis=axis)
        pad_shape = list(x.shape)
        pad_shape[axis] = s
        pad = jnp.full(pad_shape, identity, dtype=x.dtype)
        return jnp.concatenate([body, pad], axis=axis)


def _ks_prefix(a, b, axis):
    n = a.shape[axis]
    s = 1
    while s < n:
        ap = _shift(a, axis, s, True, 1.0)
        bp = _shift(b, axis, s, True, 0.0)
        a, b = ap * a, a * bp + b
        s *= 2
    return a, b


def _ks_suffix(a, b, axis):
    n = a.shape[axis]
    s = 1
    while s < n:
        ap = _shift(a, axis, s, False, 1.0)
        bp = _shift(b, axis, s, False, 0.0)
        a, b = a * ap, a * bp + b
        s *= 2
    return a, b


def _tree_ssm_kernel(xt_ref, wp_ref, dtw_ref, bias_ref, alog_ref, ds_ref,
                     hw_ref, hb_ref, ow_ref, ob_ref, out_ref, *, C, T, D):
    L = C * T
    XT = xt_ref[0]
    wp = wp_ref[...]
    dtw = dtw_ref[...]

    xdbl = lax.dot_general(XT, wp, (((1,), (1,)), ((), ())),
                           preferred_element_type=jnp.float32)
    R = dtw.shape[1]
    dts6 = xdbl[:, 0:R]
    Bs = xdbl[:, R:R + 1]
    Cs = xdbl[:, R + 1:R + 2]
    dts = lax.dot_general(dts6, dtw, (((1,), (1,)), ((), ())),
                          preferred_element_type=jnp.float32)

    sp = jax.nn.softplus(dts + bias_ref[...])
    A = -jnp.exp(alog_ref[...])
    w = jnp.exp(sp * A)
    f = sp * Bs * XT

    W3 = w.reshape(C, T, D)
    F3 = f.reshape(C, T, D)

    CA, CB = _ks_prefix(W3, F3, axis=1)
    A2 = CA[:, T - 1, :]
    B2 = CB[:, T - 1, :]
    _, GB = _ks_prefix(A2, B2, axis=0)
    G = _shift(GB, 0, 1, True, 0.0)
    FWD = CB + CA * G[:, None, :]

    wn = _shift(w, 0, 1, False, 0.0)
    WN3 = wn.reshape(C, T, D)
    SA, SB = _ks_suffix(WN3, F3, axis=1)
    A2r = SA[:, 0, :]
    B2r = SB[:, 0, :]
    _, HB = _ks_suffix(A2r, B2r, axis=0)
    Gr = _shift(HB, 0, 1, False, 0.0)
    BWD = SB + SA * Gr[:, None, :]

    FT = FWD.reshape(L, D) + BWD.reshape(L, D) - f

    eps = 1e-5
    mu = jnp.mean(FT, axis=-1, keepdims=True)
    var = jnp.mean((FT - mu) ** 2, axis=-1, keepdims=True)
    out = (FT - mu) * lax.rsqrt(var + eps) * hw_ref[...] + hb_ref[...]

    y = out * Cs
    y = y + ds_ref[...] * XT

    mu2 = jnp.mean(y, axis=-1, keepdims=True)
    var2 = jnp.mean((y - mu2) ** 2, axis=-1, keepdims=True)
    y = (y - mu2) * lax.rsqrt(var2 + eps) * ow_ref[...] + ob_ref[...]

    out_ref[0] = y


def kernel(x, x_proj_weight, dt_projs_weight, dt_projs_bias, A_logs, Ds,
           h_norm_w, h_norm_b, out_norm_w, out_norm_b):
    B, D, H, W = x.shape
    L = H * W
    C = 56
    T = L // C
    assert C * T == L

    xt = jnp.transpose(x.reshape(B, D, L), (0, 2, 1)).astype(jnp.float32)
    wp = x_proj_weight[0].astype(jnp.float32)
    dtw = dt_projs_weight[0].astype(jnp.float32)
    bias = dt_projs_bias.reshape(1, D).astype(jnp.float32)
    alog = A_logs.reshape(1, D).astype(jnp.float32)
    ds = Ds.reshape(1, D).astype(jnp.float32)
    hw = h_norm_w.reshape(1, D).astype(jnp.float32)
    hb = h_norm_b.reshape(1, D).astype(jnp.float32)
    ow = out_norm_w.reshape(1, D).astype(jnp.float32)
    ob = out_norm_b.reshape(1, D).astype(jnp.float32)

    import functools
    body = functools.partial(_tree_ssm_kernel, C=C, T=T, D=D)

    vec = pl.BlockSpec((1, D), lambda b: (0, 0))
    y = pl.pallas_call(
        body,
        grid=(B,),
        in_specs=[
            pl.BlockSpec((1, L, D), lambda b: (b, 0, 0)),
            pl.BlockSpec(wp.shape, lambda b: (0, 0)),
            pl.BlockSpec(dtw.shape, lambda b: (0, 0)),
            vec, vec, vec, vec, vec, vec, vec,
        ],
        out_specs=pl.BlockSpec((1, L, D), lambda b: (b, 0, 0)),
        out_shape=jax.ShapeDtypeStruct((B, L, D), jnp.float32),
    )(xt, wp, dtw, bias, alog, ds, hw, hb, ow, ob)

    return y.reshape(B, H, W, D).astype(x.dtype)
